# Initial kernel scaffold; baseline (speedup 1.0000x reference)
import jax
import jax.numpy as jnp
from jax.experimental import pallas as pl


def kernel(user_ids, item_ids, user_table, item_table):
    raise NotImplementedError("write your pallas kernel here")


# speedup vs baseline: 1.2169x; 1.2169x over previous
import functools

import jax
import jax.numpy as jnp
from jax import lax
from jax.experimental import pallas as pl
from jax.experimental.pallas import tpu as pltpu
from jax.experimental.pallas import tpu_sc as plsc

NC = 2
NS = 16
L = 16
NW = NC * NS

BATCH = 16384
D = 128
BPW = BATCH // NW
CHUNK = 128
NCHUNK = BPW // CHUNK

_mesh = plsc.VectorSubcoreMesh(
    core_axis_name="c", subcore_axis_name="s", num_cores=NC, num_subcores=NS
)


@functools.partial(
    pl.kernel,
    out_type=jax.ShapeDtypeStruct((NW, BPW), jnp.float32),
    mesh=_mesh,
    compiler_params=pltpu.CompilerParams(needs_layout_passes=False),
    scratch_types=[
        pltpu.VMEM((NCHUNK, CHUNK), jnp.int32),
        pltpu.VMEM((NCHUNK, CHUNK), jnp.int32),
        pltpu.VMEM((CHUNK, D), jnp.float32),
        pltpu.VMEM((CHUNK, D), jnp.float32),
        pltpu.VMEM((CHUNK, D), jnp.float32),
        pltpu.VMEM((CHUNK, D), jnp.float32),
        pltpu.VMEM((BPW,), jnp.float32),
        pltpu.SemaphoreType.DMA,
        pltpu.SemaphoreType.DMA,
        pltpu.SemaphoreType.DMA,
        pltpu.SemaphoreType.DMA,
    ],
)
def _mf_sc_kernel(
    uids_hbm, iids_hbm, utab_hbm, itab_hbm, out_hbm,
    uidx, iidx, ubuf0, ubuf1, ibuf0, ibuf1, outv,
    usem0, usem1, isem0, isem1,
):
    wid = lax.axis_index("s") * NC + lax.axis_index("c")
    pltpu.sync_copy(uids_hbm.at[wid], uidx)
    pltpu.sync_copy(iids_hbm.at[wid], iidx)

    ubufs = (ubuf0, ubuf1)
    ibufs = (ibuf0, ibuf1)
    usems = (usem0, usem1)
    isems = (isem0, isem1)

    def start(c):
        p = c % 2
        hu = pltpu.async_copy(utab_hbm.at[uidx.at[c]], ubufs[p], usems[p])
        hi = pltpu.async_copy(itab_hbm.at[iidx.at[c]], ibufs[p], isems[p])
        return hu, hi

    lane = lax.iota(jnp.int32, L)
    mask_last = lane == (L - 1)
    UNROLL = 4
    handles = [None, None]
    handles[0] = start(0)
    for c in range(NCHUNK):
        p = c % 2
        if c + 1 < NCHUNK:
            handles[(c + 1) % 2] = start(c + 1)
        hu, hi = handles[p]
        hu.wait()
        hi.wait()
        ub = ubufs[p]
        ib = ibufs[p]

        def group(q, _, ub=ub, ib=ib, base=c * CHUNK):
            for j in range(UNROLL):
                r = q * UNROLL + j
                prods = [
                    ub[r, pl.ds(k * L, L)] * ib[r, pl.ds(k * L, L)]
                    for k in range(D // L)
                ]
                while len(prods) > 1:
                    prods = [
                        prods[i] + prods[i + 1] if i + 1 < len(prods) else prods[i]
                        for i in range(0, len(prods), 2)
                    ]
                s = plsc.cumsum(prods[0])
                idx = jnp.full((L,), base + r, jnp.int32)
                plsc.store_scatter(outv, [idx], s, mask=mask_last)
            return 0

        lax.fori_loop(0, CHUNK // UNROLL, group, 0)

    pltpu.sync_copy(outv, out_hbm.at[wid])


def kernel(user_ids, item_ids, user_table, item_table):
    uids = user_ids.reshape(NW, NCHUNK, CHUNK)
    iids = item_ids.reshape(NW, NCHUNK, CHUNK)
    out = _mf_sc_kernel(uids, iids, user_table, item_table)
    return out.reshape(BATCH)


# speedup vs baseline: 1.3447x; 1.1050x over previous
import functools

import jax
import jax.numpy as jnp
from jax import lax
from jax.experimental import pallas as pl
from jax.experimental.pallas import tpu as pltpu
from jax.experimental.pallas import tpu_sc as plsc

NC = 2
NS = 16
L = 16
NW = NC * NS

BATCH = 16384
D = 128
BPW = BATCH // NW
CHUNK = 128
NCHUNK = BPW // CHUNK

_mesh = plsc.VectorSubcoreMesh(
    core_axis_name="c", subcore_axis_name="s", num_cores=NC, num_subcores=NS
)


@functools.partial(
    pl.kernel,
    out_type=jax.ShapeDtypeStruct((NW, BPW), jnp.float32),
    mesh=_mesh,
    compiler_params=pltpu.CompilerParams(needs_layout_passes=False),
    scratch_types=[
        pltpu.VMEM((NCHUNK, CHUNK), jnp.int32),
        pltpu.VMEM((NCHUNK, CHUNK), jnp.int32),
        pltpu.VMEM((CHUNK, D), jnp.float32),
        pltpu.VMEM((CHUNK, D), jnp.float32),
        pltpu.VMEM((CHUNK, D), jnp.float32),
        pltpu.VMEM((CHUNK, D), jnp.float32),
        pltpu.VMEM((BPW,), jnp.float32),
        pltpu.VMEM((L, L + 1), jnp.float32),
        pltpu.SemaphoreType.DMA,
        pltpu.SemaphoreType.DMA,
        pltpu.SemaphoreType.DMA,
        pltpu.SemaphoreType.DMA,
    ],
)
def _mf_sc_kernel(
    uids_hbm, iids_hbm, utab_hbm, itab_hbm, out_hbm,
    uidx, iidx, ubuf0, ubuf1, ibuf0, ibuf1, outv, part,
    usem0, usem1, isem0, isem1,
):
    wid = lax.axis_index("s") * NC + lax.axis_index("c")
    pltpu.sync_copy(uids_hbm.at[wid], uidx)
    pltpu.sync_copy(iids_hbm.at[wid], iidx)

    ubufs = (ubuf0, ubuf1)
    ibufs = (ibuf0, ibuf1)
    usems = (usem0, usem1)
    isems = (isem0, isem1)

    def start(c):
        p = c % 2
        hu = pltpu.async_copy(utab_hbm.at[uidx.at[c]], ubufs[p], usems[p])
        hi = pltpu.async_copy(itab_hbm.at[iidx.at[c]], ibufs[p], isems[p])
        return hu, hi

    lane = lax.iota(jnp.int32, L)
    col_last = jnp.full((L,), L - 1, jnp.int32)
    handles = [None, None]
    handles[0] = start(0)
    for c in range(NCHUNK):
        p = c % 2
        if c + 1 < NCHUNK:
            handles[(c + 1) % 2] = start(c + 1)
        hu, hi = handles[p]
        hu.wait()
        hi.wait()
        ub = ubufs[p]
        ib = ibufs[p]

        def group(g, _, ub=ub, ib=ib, base=c * CHUNK):
            SUB = 4
            for h in range(L // SUB):
                sums_by_row = []
                for j in range(SUB):
                    r = g * L + h * SUB + j
                    prods = [
                        ub[r, pl.ds(k * L, L)] * ib[r, pl.ds(k * L, L)]
                        for k in range(D // L)
                    ]
                    while len(prods) > 1:
                        prods = [
                            prods[i] + prods[i + 1] if i + 1 < len(prods) else prods[i]
                            for i in range(0, len(prods), 2)
                        ]
                    sums_by_row.append(plsc.cumsum(prods[0]))
                for j in range(SUB):
                    part[h * SUB + j, pl.ds(0, L)] = sums_by_row[j]
            sums = plsc.load_gather(part, [lane, col_last])
            outv[pl.ds(base + g * L, L)] = sums
            return 0

        lax.fori_loop(0, CHUNK // L, group, 0)

    pltpu.sync_copy(outv, out_hbm.at[wid])


def kernel(user_ids, item_ids, user_table, item_table):
    uids = user_ids.reshape(NW, NCHUNK, CHUNK)
    iids = item_ids.reshape(NW, NCHUNK, CHUNK)
    out = _mf_sc_kernel(uids, iids, user_table, item_table)
    return out.reshape(BATCH)
